# baseline (speedup 1.0000x reference)
import functools

import jax
import jax.numpy as jnp
from jax import lax
from jax.experimental import pallas as pl
from jax.experimental.pallas import tpu as pltpu
from jax.experimental.pallas import tpu_sc as plsc

N = 100000
E = 1600000
H = 32
NSORT = 131072
SROWS = NSORT // 128

NW = 32
CH = 512
NCHUNK = E // CH


def _tables_body(emb_ref, f_ref, w1_ref, b1_ref, p_ref, q_ref):
    emb = emb_ref[...]
    f = f_ref[...]
    w1 = w1_ref[...]
    wa = w1[0:H, :]
    wb = w1[H:2 * H, :]
    wc = w1[2 * H:2 * H + 1, :]
    wd = w1[2 * H + 1:2 * H + 2, :]
    b1 = b1_ref[...]
    p_ref[...] = (jnp.dot(emb, wa, preferred_element_type=jnp.float32)
                  + f * wc).astype(jnp.bfloat16)
    q_ref[...] = (jnp.dot(emb, wb, preferred_element_type=jnp.float32)
                  + f * wd + b1).astype(jnp.bfloat16)


def _node_tables(emb, f2d, w1, b1):
    blk = 5000
    grid = N // blk
    return pl.pallas_call(
        _tables_body,
        grid=(grid,),
        in_specs=[
            pl.BlockSpec((blk, H), lambda i: (i, 0)),
            pl.BlockSpec((blk, 1), lambda i: (i, 0)),
            pl.BlockSpec((2 * H + 2, H), lambda i: (0, 0)),
            pl.BlockSpec((1, H), lambda i: (0, 0)),
        ],
        out_specs=[
            pl.BlockSpec((blk, H), lambda i: (i, 0)),
            pl.BlockSpec((blk, H), lambda i: (i, 0)),
        ],
        out_shape=[
            jax.ShapeDtypeStruct((N, H), jnp.bfloat16),
            jax.ShapeDtypeStruct((N, H), jnp.bfloat16),
        ],
    )(emb, f2d, w1, b1)


NSUB = CH // 128


def _edge_gather_body(p_hbm, q_hbm, ei_hbm, outa_hbm, outb_hbm,
                      idx0, idx1, abuf0, abuf1, bbuf0, bbuf1,
                      semi0, semi1, semg0, semg1, semw0, semw1):
    wid = lax.axis_index("s") * 2 + lax.axis_index("c")
    base_chunks = NCHUNK // NW
    extra = NCHUNK - base_chunks * NW
    nt = jnp.where(wid < extra, base_chunks + 1, base_chunks)
    idx = (idx0, idx1)
    abuf = (abuf0, abuf1)
    bbuf = (bbuf0, bbuf1)
    semi = (semi0, semi1)
    semg = (semg0, semg1)
    semw = (semw0, semw1)

    def cbase(t):
        return (t * NW + wid) * CH

    def fire_gathers(t, b):
        for j in range(NSUB):
            js = pl.ds(j * 128, 128)
            pltpu.async_copy(p_hbm.at[idx[b].at[0, js]], abuf[b].at[js],
                             semg[b])
            pltpu.async_copy(q_hbm.at[idx[b].at[1, js]], bbuf[b].at[js],
                             semg[b])

    def wait_gathers(t, b):
        for j in range(NSUB):
            js = pl.ds(j * 128, 128)
            pltpu.make_async_copy(p_hbm.at[idx[b].at[0, js]],
                                  abuf[b].at[js], semg[b]).wait()
            pltpu.make_async_copy(q_hbm.at[idx[b].at[1, js]],
                                  bbuf[b].at[js], semg[b]).wait()

    def fire_idx(t, b):
        pltpu.async_copy(ei_hbm.at[:, pl.ds(cbase(t), CH)], idx[b], semi[b])

    def fire_write(t, b):
        pltpu.async_copy(abuf[b], outa_hbm.at[pl.ds(cbase(t), CH)], semw[b])
        pltpu.async_copy(bbuf[b], outb_hbm.at[pl.ds(cbase(t), CH)], semw[b])

    def wait_write(b):
        pltpu.make_async_copy(abuf[b], outa_hbm.at[pl.ds(0, CH)],
                              semw[b]).wait()
        pltpu.make_async_copy(bbuf[b], outb_hbm.at[pl.ds(0, CH)],
                              semw[b]).wait()

    pltpu.sync_copy(ei_hbm.at[:, pl.ds(cbase(0), CH)], idx[0])
    fire_gathers(0, 0)

    @pl.when(nt > 1)
    def _():
        fire_idx(1, 1)

    def body(tt, carry):
        for b in (0, 1):
            t = tt * 2 + b

            @pl.when(t < nt)
            def _():
                @pl.when(t >= 1)
                def _():
                    wait_write(1 - b)

                @pl.when(t + 1 < nt)
                def _():
                    pltpu.make_async_copy(
                        ei_hbm.at[:, pl.ds(0, CH)], idx[1 - b],
                        semi[1 - b]).wait()
                    fire_gathers(t + 1, 1 - b)

                wait_gathers(t, b)

                @pl.when(t + 2 < nt)
                def _():
                    fire_idx(t + 2, b)

                fire_write(t, b)

        return carry

    lax.fori_loop(0, (base_chunks + 2) // 2, body, 0)

    p_last = (nt - 1) % 2

    @pl.when(p_last == 0)
    def _():
        wait_write(0)

    @pl.when(p_last == 1)
    def _():
        wait_write(1)


def _edge_gather(p_tab, q_tab, edge_index):
    mesh = plsc.VectorSubcoreMesh(core_axis_name="c", subcore_axis_name="s")
    fn = functools.partial(
        pl.kernel,
        out_type=[
            jax.ShapeDtypeStruct((E, H), jnp.bfloat16),
            jax.ShapeDtypeStruct((E, H), jnp.bfloat16),
        ],
        mesh=mesh,
        compiler_params=pltpu.CompilerParams(use_tc_tiling_on_sc=False),
        scratch_types=[
            pltpu.VMEM((2, CH), jnp.int32),
            pltpu.VMEM((2, CH), jnp.int32),
            pltpu.VMEM((CH, H), jnp.bfloat16),
            pltpu.VMEM((CH, H), jnp.bfloat16),
            pltpu.VMEM((CH, H), jnp.bfloat16),
            pltpu.VMEM((CH, H), jnp.bfloat16),
            pltpu.SemaphoreType.DMA,
            pltpu.SemaphoreType.DMA,
            pltpu.SemaphoreType.DMA,
            pltpu.SemaphoreType.DMA,
            pltpu.SemaphoreType.DMA,
            pltpu.SemaphoreType.DMA,
        ],
    )(_edge_gather_body)
    return fn(p_tab, q_tab, edge_index)


def _tail_body(ha_ref, hb_ref, w2_ref, b2_ref, w3_ref, b3_ref, out_ref):
    h = jnp.maximum(ha_ref[...] + hb_ref[...], jnp.bfloat16(0.0))
    h2 = jnp.maximum(
        jnp.dot(h, w2_ref[...].astype(jnp.bfloat16),
                preferred_element_type=jnp.float32)
        + b2_ref[...], 0.0)
    s = jnp.sum(h2 * w3_ref[...], axis=1) + b3_ref[0, 0]
    out_ref[...] = s


def _mlp_tail(h1a, h1b, w2, b2, w3t, b3):
    blk = 16384
    grid = pl.cdiv(E, blk)
    return pl.pallas_call(
        _tail_body,
        grid=(grid,),
        in_specs=[
            pl.BlockSpec((blk, H), lambda i: (i, 0)),
            pl.BlockSpec((blk, H), lambda i: (i, 0)),
            pl.BlockSpec((H, H), lambda i: (0, 0)),
            pl.BlockSpec((1, H), lambda i: (0, 0)),
            pl.BlockSpec((1, H), lambda i: (0, 0)),
            pl.BlockSpec((1, 1), lambda i: (0, 0)),
        ],
        out_specs=pl.BlockSpec((blk,), lambda i: (i,)),
        out_shape=jax.ShapeDtypeStruct((E,), jnp.float32),
    )(h1a, h1b, w2, b2, w3t, b3)


STILE = 32
SNT = SROWS // STILE


def _bitonic_body(k_in_ref, out_ref, kref, vref):
    riota = lax.broadcasted_iota(jnp.int32, (STILE, 128), 0)
    liota = lax.broadcasted_iota(jnp.int32, (STILE, 128), 1)

    def init(t, c):
        kref[pl.ds(t * STILE, STILE), :] = k_in_ref[pl.ds(t * STILE, STILE), :]
        vref[pl.ds(t * STILE, STILE), :] = (riota + t * STILE) * 128 + liota
        return c

    lax.fori_loop(0, SNT, init, 0)

    def lexlt(ak, av, bk, bv):
        return (ak < bk) | ((ak == bk) & (av < bv))

    def stage_local(k, d_list):
        def body(t, c):
            rs = pl.ds(t * STILE, STILE)
            kt = kref[rs, :]
            vt = vref[rs, :]
            for d in d_list:
                if d < 128:
                    bit = (liota & d) == 0
                    pk = jnp.where(bit, jnp.roll(kt, -d, axis=1),
                                   jnp.roll(kt, d, axis=1))
                    pv = jnp.where(bit, jnp.roll(vt, -d, axis=1),
                                   jnp.roll(vt, d, axis=1))
                else:
                    e = d // 128
                    bit = (riota & e) == 0
                    pk = jnp.where(bit, jnp.roll(kt, -e, axis=0),
                                   jnp.roll(kt, e, axis=0))
                    pv = jnp.where(bit, jnp.roll(vt, -e, axis=0),
                                   jnp.roll(vt, e, axis=0))
                if k < 128:
                    up = (liota & k) == 0
                elif k // 128 < STILE:
                    up = (riota & (k // 128)) == 0
                else:
                    ups = ((t * STILE) & (k // 128)) == 0
                    up = jnp.full((STILE, 128), False) | ups
                take_min = bit == up
                sel_self = take_min == lexlt(kt, vt, pk, pv)
                kt = jnp.where(sel_self, kt, pk)
                vt = jnp.where(sel_self, vt, pv)
            kref[rs, :] = kt
            vref[rs, :] = vt
            return c

        lax.fori_loop(0, SNT, body, 0)

    def stage_pair(k, d):
        e = d // 128

        def body(t, c):
            m = t * STILE
            g = m // e
            lo = g * 2 * e + (m - g * e)
            lo_s = pl.ds(lo, STILE)
            hi_s = pl.ds(lo + e, STILE)
            ak, av = kref[lo_s, :], vref[lo_s, :]
            bk, bv = kref[hi_s, :], vref[hi_s, :]
            sel = lexlt(ak, av, bk, bv)
            mink = jnp.where(sel, ak, bk)
            minv = jnp.where(sel, av, bv)
            maxk = jnp.where(sel, bk, ak)
            maxv = jnp.where(sel, bv, av)
            up = jnp.full((STILE, 128), False) | ((lo & (k // 128)) == 0)
            kref[lo_s, :] = jnp.where(up, mink, maxk)
            vref[lo_s, :] = jnp.where(up, minv, maxv)
            kref[hi_s, :] = jnp.where(up, maxk, mink)
            vref[hi_s, :] = jnp.where(up, maxv, minv)
            return c

        lax.fori_loop(0, SNT // 2, body, 0)

    k = 2
    while k <= NSORT:
        d = k // 2
        local_ds = []
        while d >= 1:
            if d < 128 * STILE:
                local_ds.append(d)
            else:
                stage_pair(k, d)
            d //= 2
        stage_local(k, local_ds)
        k *= 2

    def wout(t, c):
        out_ref[pl.ds(t * STILE, STILE), :] = vref[pl.ds(t * STILE, STILE), :]
        return c

    lax.fori_loop(0, SNT, wout, 0)


def _bitonic_argsort(keys_padded_2d):
    return pl.pallas_call(
        _bitonic_body,
        out_shape=jax.ShapeDtypeStruct((SROWS, 128), jnp.int32),
        scratch_shapes=[
            pltpu.VMEM((SROWS, 128), jnp.float32),
            pltpu.VMEM((SROWS, 128), jnp.int32),
        ],
    )(keys_padded_2d)


NPOS = 2048


def _cand_body(sorted_hbm, pos_hbm, out_hbm, pos_v, val_v, sem):
    wid = lax.axis_index("s") * 2 + lax.axis_index("c")

    @pl.when(wid < NPOS // CH)
    def _():
        base = wid * CH
        pltpu.sync_copy(pos_hbm.at[pl.ds(base, CH)], pos_v)
        pltpu.async_copy(sorted_hbm.at[pos_v], val_v, sem).wait()
        pltpu.sync_copy(val_v, out_hbm.at[pl.ds(base, CH)])


def _gather_candidates(sorted_flat, pos):
    mesh = plsc.VectorSubcoreMesh(core_axis_name="c", subcore_axis_name="s")
    fn = functools.partial(
        pl.kernel,
        out_type=jax.ShapeDtypeStruct((NPOS,), jnp.int32),
        mesh=mesh,
        compiler_params=pltpu.CompilerParams(use_tc_tiling_on_sc=False),
        scratch_types=[
            pltpu.VMEM((CH,), jnp.int32),
            pltpu.VMEM((CH,), jnp.int32),
            pltpu.SemaphoreType.DMA,
        ],
    )(_cand_body)
    return fn(sorted_flat, pos)


def _candidate_positions():
    key = jax.random.key(42)
    k1, k2 = jax.random.split(key)
    idx1 = jax.random.randint(k1, (1000,), 0, N // 2, dtype=jnp.int32)
    idx2 = jax.random.randint(k2, (1000,), N // 2, N, dtype=jnp.int32)
    pad = jnp.zeros((NPOS - 2000,), jnp.int32)
    return jnp.concatenate([idx1, idx2, pad])


def kernel(node_embeddings, edge_index, num_nodes, fiedler_vector,
           W1, b1, W2, b2, W3, b3):
    f2d = fiedler_vector[:, None]
    p_tab, q_tab = _node_tables(node_embeddings, f2d, W1, b1[None, :])

    h1a, h1b = _edge_gather(p_tab, q_tab, edge_index)
    scores = _mlp_tail(h1a, h1b, W2, b2[None, :], W3.T, b3[None, :])

    keys_padded = jnp.concatenate(
        [fiedler_vector,
         jnp.full((NSORT - N,), jnp.inf, jnp.float32)]).reshape(SROWS, 128)
    sorted_idx = _bitonic_argsort(keys_padded).reshape(NSORT)

    pos = _candidate_positions()
    cand_flat = _gather_candidates(sorted_idx, pos)
    candidate_edges = jnp.stack([cand_flat[:1000], cand_flat[1000:2000]])
    return scores, candidate_edges


# speedup vs baseline: 1.3290x; 1.3290x over previous
import functools

import jax
import jax.numpy as jnp
from jax import lax
from jax.experimental import pallas as pl
from jax.experimental.pallas import tpu as pltpu
from jax.experimental.pallas import tpu_sc as plsc

N = 100000
E = 1600000
H = 32
NSORT = 131072
SROWS = NSORT // 128

NW = 32
CH = 512
NCHUNK = E // CH


def _tables_body(emb_ref, f_ref, w1_ref, b1_ref, p_ref, q_ref):
    emb = emb_ref[...]
    f = f_ref[...]
    w1 = w1_ref[...]
    wa = w1[0:H, :]
    wb = w1[H:2 * H, :]
    wc = w1[2 * H:2 * H + 1, :]
    wd = w1[2 * H + 1:2 * H + 2, :]
    b1 = b1_ref[...]
    p_ref[...] = (jnp.dot(emb, wa, preferred_element_type=jnp.float32)
                  + f * wc)
    q_ref[...] = (jnp.dot(emb, wb, preferred_element_type=jnp.float32)
                  + f * wd + b1)


def _node_tables(emb, f2d, w1, b1):
    blk = 5000
    grid = N // blk
    return pl.pallas_call(
        _tables_body,
        grid=(grid,),
        in_specs=[
            pl.BlockSpec((blk, H), lambda i: (i, 0)),
            pl.BlockSpec((blk, 1), lambda i: (i, 0)),
            pl.BlockSpec((2 * H + 2, H), lambda i: (0, 0)),
            pl.BlockSpec((1, H), lambda i: (0, 0)),
        ],
        out_specs=[
            pl.BlockSpec((blk, H), lambda i: (i, 0)),
            pl.BlockSpec((blk, H), lambda i: (i, 0)),
        ],
        out_shape=[
            jax.ShapeDtypeStruct((N, H), jnp.float32),
            jax.ShapeDtypeStruct((N, H), jnp.float32),
        ],
    )(emb, f2d, w1, b1)


NSUB = CH // 128


def _edge_gather_body(p_hbm, q_hbm, ei_hbm, outa_hbm, outb_hbm,
                      idx0, idx1, abuf0, abuf1, bbuf0, bbuf1,
                      semi0, semi1, semg0, semg1, semw0, semw1):
    wid = lax.axis_index("s") * 2 + lax.axis_index("c")
    base_chunks = NCHUNK // NW
    extra = NCHUNK - base_chunks * NW
    nt = jnp.where(wid < extra, base_chunks + 1, base_chunks)
    idx = (idx0, idx1)
    abuf = (abuf0, abuf1)
    bbuf = (bbuf0, bbuf1)
    semi = (semi0, semi1)
    semg = (semg0, semg1)
    semw = (semw0, semw1)

    def cbase(t):
        return (t * NW + wid) * CH

    def fire_gathers(t, b):
        for j in range(NSUB):
            js = pl.ds(j * 128, 128)
            pltpu.async_copy(p_hbm.at[idx[b].at[0, js]], abuf[b].at[js],
                             semg[b])
            pltpu.async_copy(q_hbm.at[idx[b].at[1, js]], bbuf[b].at[js],
                             semg[b])

    def wait_gathers(t, b):
        for j in range(NSUB):
            js = pl.ds(j * 128, 128)
            pltpu.make_async_copy(p_hbm.at[idx[b].at[0, js]],
                                  abuf[b].at[js], semg[b]).wait()
            pltpu.make_async_copy(q_hbm.at[idx[b].at[1, js]],
                                  bbuf[b].at[js], semg[b]).wait()

    def fire_idx(t, b):
        pltpu.async_copy(ei_hbm.at[:, pl.ds(cbase(t), CH)], idx[b], semi[b])

    def fire_write(t, b):
        pltpu.async_copy(abuf[b], outa_hbm.at[pl.ds(cbase(t), CH)], semw[b])
        pltpu.async_copy(bbuf[b], outb_hbm.at[pl.ds(cbase(t), CH)], semw[b])

    def wait_write(b):
        pltpu.make_async_copy(abuf[b], outa_hbm.at[pl.ds(0, CH)],
                              semw[b]).wait()
        pltpu.make_async_copy(bbuf[b], outb_hbm.at[pl.ds(0, CH)],
                              semw[b]).wait()

    pltpu.sync_copy(ei_hbm.at[:, pl.ds(cbase(0), CH)], idx[0])
    fire_gathers(0, 0)

    @pl.when(nt > 1)
    def _():
        fire_idx(1, 1)

    def body(tt, carry):
        for b in (0, 1):
            t = tt * 2 + b

            @pl.when(t < nt)
            def _():
                @pl.when(t >= 1)
                def _():
                    wait_write(1 - b)

                @pl.when(t + 1 < nt)
                def _():
                    pltpu.make_async_copy(
                        ei_hbm.at[:, pl.ds(0, CH)], idx[1 - b],
                        semi[1 - b]).wait()
                    fire_gathers(t + 1, 1 - b)

                wait_gathers(t, b)

                @pl.when(t + 2 < nt)
                def _():
                    fire_idx(t + 2, b)

                fire_write(t, b)

        return carry

    lax.fori_loop(0, (base_chunks + 2) // 2, body, 0)

    p_last = (nt - 1) % 2

    @pl.when(p_last == 0)
    def _():
        wait_write(0)

    @pl.when(p_last == 1)
    def _():
        wait_write(1)


def _edge_gather(p_tab, q_tab, edge_index):
    mesh = plsc.VectorSubcoreMesh(core_axis_name="c", subcore_axis_name="s")
    fn = functools.partial(
        pl.kernel,
        out_type=[
            jax.ShapeDtypeStruct((E, H), jnp.float32),
            jax.ShapeDtypeStruct((E, H), jnp.float32),
        ],
        mesh=mesh,
        compiler_params=pltpu.CompilerParams(use_tc_tiling_on_sc=False),
        scratch_types=[
            pltpu.VMEM((2, CH), jnp.int32),
            pltpu.VMEM((2, CH), jnp.int32),
            pltpu.VMEM((CH, H), jnp.float32),
            pltpu.VMEM((CH, H), jnp.float32),
            pltpu.VMEM((CH, H), jnp.float32),
            pltpu.VMEM((CH, H), jnp.float32),
            pltpu.SemaphoreType.DMA,
            pltpu.SemaphoreType.DMA,
            pltpu.SemaphoreType.DMA,
            pltpu.SemaphoreType.DMA,
            pltpu.SemaphoreType.DMA,
            pltpu.SemaphoreType.DMA,
        ],
    )(_edge_gather_body)
    return fn(p_tab, q_tab, edge_index)


def _tail_body(ha_ref, hb_ref, w2_ref, b2c_ref, w3c_ref, b3_ref, out_ref):
    h = jnp.maximum(ha_ref[...] + hb_ref[...], 0.0)
    h2t = jnp.maximum(
        lax.dot_general(w2_ref[...], h,
                        (((0,), (1,)), ((), ())),
                        preferred_element_type=jnp.float32)
        + b2c_ref[...], 0.0)
    s = jnp.sum(h2t * w3c_ref[...], axis=0, keepdims=True) + b3_ref[0, 0]
    out_ref[...] = s[None]


def _mlp_tail(h1a, h1b, w2, b2c, w3c, b3):
    blk = 16000
    grid = E // blk
    out = pl.pallas_call(
        _tail_body,
        grid=(grid,),
        in_specs=[
            pl.BlockSpec((blk, H), lambda i: (i, 0)),
            pl.BlockSpec((blk, H), lambda i: (i, 0)),
            pl.BlockSpec((H, H), lambda i: (0, 0)),
            pl.BlockSpec((H, 1), lambda i: (0, 0)),
            pl.BlockSpec((H, 1), lambda i: (0, 0)),
            pl.BlockSpec((1, 1), lambda i: (0, 0)),
        ],
        out_specs=pl.BlockSpec((1, 1, blk), lambda i: (i, 0, 0)),
        out_shape=jax.ShapeDtypeStruct((grid, 1, blk), jnp.float32),
    )(h1a, h1b, w2, b2c, w3c, b3)
    return out.reshape(E)


STILE = 32
SNT = SROWS // STILE


def _bitonic_body(k_in_ref, out_ref, kref, vref):
    riota = lax.broadcasted_iota(jnp.int32, (STILE, 128), 0)
    liota = lax.broadcasted_iota(jnp.int32, (STILE, 128), 1)

    def init(t, c):
        kref[pl.ds(t * STILE, STILE), :] = k_in_ref[pl.ds(t * STILE, STILE), :]
        vref[pl.ds(t * STILE, STILE), :] = (riota + t * STILE) * 128 + liota
        return c

    lax.fori_loop(0, SNT, init, 0)

    def lexlt(ak, av, bk, bv):
        return (ak < bk) | ((ak == bk) & (av < bv))

    def stage_local(k, d_list):
        def body(t, c):
            rs = pl.ds(t * STILE, STILE)
            kt = kref[rs, :]
            vt = vref[rs, :]
            for d in d_list:
                if d < 128:
                    bit = (liota & d) == 0
                    pk = jnp.where(bit, jnp.roll(kt, -d, axis=1),
                                   jnp.roll(kt, d, axis=1))
                    pv = jnp.where(bit, jnp.roll(vt, -d, axis=1),
                                   jnp.roll(vt, d, axis=1))
                else:
                    e = d // 128
                    bit = (riota & e) == 0
                    pk = jnp.where(bit, jnp.roll(kt, -e, axis=0),
                                   jnp.roll(kt, e, axis=0))
                    pv = jnp.where(bit, jnp.roll(vt, -e, axis=0),
                                   jnp.roll(vt, e, axis=0))
                if k < 128:
                    up = (liota & k) == 0
                elif k // 128 < STILE:
                    up = (riota & (k // 128)) == 0
                else:
                    ups = ((t * STILE) & (k // 128)) == 0
                    up = jnp.full((STILE, 128), False) | ups
                take_min = bit == up
                sel_self = take_min == lexlt(kt, vt, pk, pv)
                kt = jnp.where(sel_self, kt, pk)
                vt = jnp.where(sel_self, vt, pv)
            kref[rs, :] = kt
            vref[rs, :] = vt
            return c

        lax.fori_loop(0, SNT, body, 0)

    def stage_pair(k, d):
        e = d // 128

        def body(t, c):
            m = t * STILE
            g = m // e
            lo = g * 2 * e + (m - g * e)
            lo_s = pl.ds(lo, STILE)
            hi_s = pl.ds(lo + e, STILE)
            ak, av = kref[lo_s, :], vref[lo_s, :]
            bk, bv = kref[hi_s, :], vref[hi_s, :]
            sel = lexlt(ak, av, bk, bv)
            mink = jnp.where(sel, ak, bk)
            minv = jnp.where(sel, av, bv)
            maxk = jnp.where(sel, bk, ak)
            maxv = jnp.where(sel, bv, av)
            up = jnp.full((STILE, 128), False) | ((lo & (k // 128)) == 0)
            kref[lo_s, :] = jnp.where(up, mink, maxk)
            vref[lo_s, :] = jnp.where(up, minv, maxv)
            kref[hi_s, :] = jnp.where(up, maxk, mink)
            vref[hi_s, :] = jnp.where(up, maxv, minv)
            return c

        lax.fori_loop(0, SNT // 2, body, 0)

    k = 2
    while k <= NSORT:
        d = k // 2
        local_ds = []
        while d >= 1:
            if d < 128 * STILE:
                local_ds.append(d)
            else:
                stage_pair(k, d)
            d //= 2
        stage_local(k, local_ds)
        k *= 2

    def wout(t, c):
        out_ref[pl.ds(t * STILE, STILE), :] = vref[pl.ds(t * STILE, STILE), :]
        return c

    lax.fori_loop(0, SNT, wout, 0)


def _bitonic_argsort(keys_padded_2d):
    return pl.pallas_call(
        _bitonic_body,
        out_shape=jax.ShapeDtypeStruct((SROWS, 128), jnp.int32),
        scratch_shapes=[
            pltpu.VMEM((SROWS, 128), jnp.float32),
            pltpu.VMEM((SROWS, 128), jnp.int32),
        ],
    )(keys_padded_2d)


NPOS = 2048


def _cand_body(sorted_hbm, pos_hbm, out_hbm, pos_v, val_v, sem):
    wid = lax.axis_index("s") * 2 + lax.axis_index("c")

    @pl.when(wid < NPOS // CH)
    def _():
        base = wid * CH
        pltpu.sync_copy(pos_hbm.at[pl.ds(base, CH)], pos_v)
        pltpu.async_copy(sorted_hbm.at[pos_v], val_v, sem).wait()
        pltpu.sync_copy(val_v, out_hbm.at[pl.ds(base, CH)])


def _gather_candidates(sorted_flat, pos):
    mesh = plsc.VectorSubcoreMesh(core_axis_name="c", subcore_axis_name="s")
    fn = functools.partial(
        pl.kernel,
        out_type=jax.ShapeDtypeStruct((NPOS,), jnp.int32),
        mesh=mesh,
        compiler_params=pltpu.CompilerParams(use_tc_tiling_on_sc=False),
        scratch_types=[
            pltpu.VMEM((CH,), jnp.int32),
            pltpu.VMEM((CH,), jnp.int32),
            pltpu.SemaphoreType.DMA,
        ],
    )(_cand_body)
    return fn(sorted_flat, pos)


def _candidate_positions():
    key = jax.random.key(42)
    k1, k2 = jax.random.split(key)
    idx1 = jax.random.randint(k1, (1000,), 0, N // 2, dtype=jnp.int32)
    idx2 = jax.random.randint(k2, (1000,), N // 2, N, dtype=jnp.int32)
    pad = jnp.zeros((NPOS - 2000,), jnp.int32)
    return jnp.concatenate([idx1, idx2, pad])


def kernel(node_embeddings, edge_index, num_nodes, fiedler_vector,
           W1, b1, W2, b2, W3, b3):
    f2d = fiedler_vector[:, None]
    p_tab, q_tab = _node_tables(node_embeddings, f2d, W1, b1[None, :])

    h1a, h1b = _edge_gather(p_tab, q_tab, edge_index)
    scores = _mlp_tail(h1a, h1b, W2, b2[:, None], W3, b3[None, :])

    keys_padded = jnp.concatenate(
        [fiedler_vector,
         jnp.full((NSORT - N,), jnp.inf, jnp.float32)]).reshape(SROWS, 128)
    sorted_idx = _bitonic_argsort(keys_padded).reshape(NSORT)

    pos = _candidate_positions()
    cand_flat = _gather_candidates(sorted_idx, pos)
    candidate_edges = jnp.stack([cand_flat[:1000], cand_flat[1000:2000]])
    return scores, candidate_edges


# speedup vs baseline: 2.2075x; 1.6610x over previous
import functools

import jax
import jax.numpy as jnp
from jax import lax
from jax.experimental import pallas as pl
from jax.experimental.pallas import tpu as pltpu
from jax.experimental.pallas import tpu_sc as plsc

N = 100000
E = 1600000
H = 32
NSORT = 131072
SROWS = NSORT // 128

NW = 32
CH = 512
NCHUNK = E // CH


def _tables_body(emb_ref, f_ref, w1_ref, b1_ref, p_ref, q_ref):
    emb = emb_ref[...]
    f = f_ref[...]
    w1 = w1_ref[...]
    wa = w1[0:H, :]
    wb = w1[H:2 * H, :]
    wc = w1[2 * H:2 * H + 1, :]
    wd = w1[2 * H + 1:2 * H + 2, :]
    b1 = b1_ref[...]
    p_ref[...] = (jnp.dot(emb, wa, preferred_element_type=jnp.float32)
                  + f * wc)
    q_ref[...] = (jnp.dot(emb, wb, preferred_element_type=jnp.float32)
                  + f * wd + b1)


def _node_tables(emb, f2d, w1, b1):
    blk = 5000
    grid = N // blk
    return pl.pallas_call(
        _tables_body,
        grid=(grid,),
        in_specs=[
            pl.BlockSpec((blk, H), lambda i: (i, 0)),
            pl.BlockSpec((blk, 1), lambda i: (i, 0)),
            pl.BlockSpec((2 * H + 2, H), lambda i: (0, 0)),
            pl.BlockSpec((1, H), lambda i: (0, 0)),
        ],
        out_specs=[
            pl.BlockSpec((blk, H), lambda i: (i, 0)),
            pl.BlockSpec((blk, H), lambda i: (i, 0)),
        ],
        out_shape=[
            jax.ShapeDtypeStruct((N, H), jnp.float32),
            jax.ShapeDtypeStruct((N, H), jnp.float32),
        ],
    )(emb, f2d, w1, b1)


NSUB = CH // 128


def _edge_gather_body(p_hbm, q_hbm, ei_hbm, outa_hbm, outb_hbm,
                      idx0, idx1, abuf0, abuf1, bbuf0, bbuf1,
                      semi0, semi1, semg0, semg1, semw0, semw1):
    wid = lax.axis_index("s") * 2 + lax.axis_index("c")
    base_chunks = NCHUNK // NW
    extra = NCHUNK - base_chunks * NW
    nt = jnp.where(wid < extra, base_chunks + 1, base_chunks)
    idx = (idx0, idx1)
    abuf = (abuf0, abuf1)
    bbuf = (bbuf0, bbuf1)
    semi = (semi0, semi1)
    semg = (semg0, semg1)
    semw = (semw0, semw1)

    def cbase(t):
        return (t * NW + wid) * CH

    def fire_gathers(t, b):
        for j in range(NSUB):
            js = pl.ds(j * 128, 128)
            pltpu.async_copy(p_hbm.at[idx[b].at[0, js]], abuf[b].at[js],
                             semg[b])
            pltpu.async_copy(q_hbm.at[idx[b].at[1, js]], bbuf[b].at[js],
                             semg[b])

    def wait_gathers(t, b):
        for j in range(NSUB):
            js = pl.ds(j * 128, 128)
            pltpu.make_async_copy(p_hbm.at[idx[b].at[0, js]],
                                  abuf[b].at[js], semg[b]).wait()
            pltpu.make_async_copy(q_hbm.at[idx[b].at[1, js]],
                                  bbuf[b].at[js], semg[b]).wait()

    def fire_idx(t, b):
        pltpu.async_copy(ei_hbm.at[:, pl.ds(cbase(t), CH)], idx[b], semi[b])

    def fire_write(t, b):
        pltpu.async_copy(abuf[b], outa_hbm.at[pl.ds(cbase(t), CH)], semw[b])
        pltpu.async_copy(bbuf[b], outb_hbm.at[pl.ds(cbase(t), CH)], semw[b])

    def wait_write(b):
        pltpu.make_async_copy(abuf[b], outa_hbm.at[pl.ds(0, CH)],
                              semw[b]).wait()
        pltpu.make_async_copy(bbuf[b], outb_hbm.at[pl.ds(0, CH)],
                              semw[b]).wait()

    pltpu.sync_copy(ei_hbm.at[:, pl.ds(cbase(0), CH)], idx[0])
    fire_gathers(0, 0)

    @pl.when(nt > 1)
    def _():
        fire_idx(1, 1)

    def body(tt, carry):
        for b in (0, 1):
            t = tt * 2 + b

            @pl.when(t < nt)
            def _():
                @pl.when(t >= 1)
                def _():
                    wait_write(1 - b)

                @pl.when(t + 1 < nt)
                def _():
                    pltpu.make_async_copy(
                        ei_hbm.at[:, pl.ds(0, CH)], idx[1 - b],
                        semi[1 - b]).wait()
                    fire_gathers(t + 1, 1 - b)

                wait_gathers(t, b)

                @pl.when(t + 2 < nt)
                def _():
                    fire_idx(t + 2, b)

                fire_write(t, b)

        return carry

    lax.fori_loop(0, (base_chunks + 2) // 2, body, 0)

    p_last = (nt - 1) % 2

    @pl.when(p_last == 0)
    def _():
        wait_write(0)

    @pl.when(p_last == 1)
    def _():
        wait_write(1)


def _edge_gather(p_tab, q_tab, edge_index):
    mesh = plsc.VectorSubcoreMesh(core_axis_name="c", subcore_axis_name="s")
    fn = functools.partial(
        pl.kernel,
        out_type=[
            jax.ShapeDtypeStruct((E, H), jnp.float32),
            jax.ShapeDtypeStruct((E, H), jnp.float32),
        ],
        mesh=mesh,
        compiler_params=pltpu.CompilerParams(use_tc_tiling_on_sc=False),
        scratch_types=[
            pltpu.VMEM((2, CH), jnp.int32),
            pltpu.VMEM((2, CH), jnp.int32),
            pltpu.VMEM((CH, H), jnp.float32),
            pltpu.VMEM((CH, H), jnp.float32),
            pltpu.VMEM((CH, H), jnp.float32),
            pltpu.VMEM((CH, H), jnp.float32),
            pltpu.SemaphoreType.DMA,
            pltpu.SemaphoreType.DMA,
            pltpu.SemaphoreType.DMA,
            pltpu.SemaphoreType.DMA,
            pltpu.SemaphoreType.DMA,
            pltpu.SemaphoreType.DMA,
        ],
    )(_edge_gather_body)
    return fn(p_tab, q_tab, edge_index)


def _tail_body(ha_ref, hb_ref, w2d_ref, b2c_ref, w3c_ref, b3_ref,
               o0_ref, o1_ref, o2_ref, o3_ref):
    h = jnp.maximum(ha_ref[...] + hb_ref[...], 0.0)
    h2t = jnp.maximum(
        lax.dot_general(w2d_ref[...], h.astype(jnp.bfloat16),
                        (((0,), (1,)), ((), ())),
                        preferred_element_type=jnp.float32)
        + b2c_ref[...], 0.0)
    hw = h2t * w3c_ref[...]
    rblk = hw.shape[1]
    s3 = jnp.sum(hw.reshape(4, H, rblk), axis=1) + b3_ref[0, 0]
    o0_ref[...] = s3[0:1, :][None]
    o1_ref[...] = s3[1:2, :][None]
    o2_ref[...] = s3[2:3, :][None]
    o3_ref[...] = s3[3:4, :][None]


def _mlp_tail(h1a, h1b, w2d, b2c, w3c, b3):
    rblk = 4000
    rows = E // 4
    grid = rows // rblk
    outs = pl.pallas_call(
        _tail_body,
        grid=(grid,),
        in_specs=[
            pl.BlockSpec((rblk, 4 * H), lambda i: (i, 0)),
            pl.BlockSpec((rblk, 4 * H), lambda i: (i, 0)),
            pl.BlockSpec((4 * H, 4 * H), lambda i: (0, 0)),
            pl.BlockSpec((4 * H, 1), lambda i: (0, 0)),
            pl.BlockSpec((4 * H, 1), lambda i: (0, 0)),
            pl.BlockSpec((1, 1), lambda i: (0, 0)),
        ],
        out_specs=[pl.BlockSpec((1, 1, rblk), lambda i: (i, 0, 0))
                   for _ in range(4)],
        out_shape=[jax.ShapeDtypeStruct((grid, 1, rblk), jnp.float32)
                   for _ in range(4)],
    )(h1a, h1b, w2d, b2c, w3c, b3)
    s = jnp.stack([o.reshape(grid, rblk) for o in outs], axis=-1)
    return s.reshape(E)


STILE = 32
SNT = SROWS // STILE


def _bitonic_body(k_in_ref, out_ref, kref, vref):
    riota = lax.broadcasted_iota(jnp.int32, (STILE, 128), 0)
    liota = lax.broadcasted_iota(jnp.int32, (STILE, 128), 1)

    def init(t, c):
        kref[pl.ds(t * STILE, STILE), :] = k_in_ref[pl.ds(t * STILE, STILE), :]
        vref[pl.ds(t * STILE, STILE), :] = (riota + t * STILE) * 128 + liota
        return c

    lax.fori_loop(0, SNT, init, 0)

    def lexlt(ak, av, bk, bv):
        return (ak < bk) | ((ak == bk) & (av < bv))

    def stage_local(k, d_list):
        def body(t, c):
            rs = pl.ds(t * STILE, STILE)
            kt = kref[rs, :]
            vt = vref[rs, :]
            for d in d_list:
                if d < 128:
                    bit = (liota & d) == 0
                    pk = jnp.where(bit, jnp.roll(kt, -d, axis=1),
                                   jnp.roll(kt, d, axis=1))
                    pv = jnp.where(bit, jnp.roll(vt, -d, axis=1),
                                   jnp.roll(vt, d, axis=1))
                else:
                    e = d // 128
                    bit = (riota & e) == 0
                    pk = jnp.where(bit, jnp.roll(kt, -e, axis=0),
                                   jnp.roll(kt, e, axis=0))
                    pv = jnp.where(bit, jnp.roll(vt, -e, axis=0),
                                   jnp.roll(vt, e, axis=0))
                if k < 128:
                    up = (liota & k) == 0
                elif k // 128 < STILE:
                    up = (riota & (k // 128)) == 0
                else:
                    ups = ((t * STILE) & (k // 128)) == 0
                    up = jnp.full((STILE, 128), False) | ups
                take_min = bit == up
                sel_self = take_min == lexlt(kt, vt, pk, pv)
                kt = jnp.where(sel_self, kt, pk)
                vt = jnp.where(sel_self, vt, pv)
            kref[rs, :] = kt
            vref[rs, :] = vt
            return c

        lax.fori_loop(0, SNT, body, 0)

    def stage_pair(k, d):
        e = d // 128

        def body(t, c):
            m = t * STILE
            g = m // e
            lo = g * 2 * e + (m - g * e)
            lo_s = pl.ds(lo, STILE)
            hi_s = pl.ds(lo + e, STILE)
            ak, av = kref[lo_s, :], vref[lo_s, :]
            bk, bv = kref[hi_s, :], vref[hi_s, :]
            sel = lexlt(ak, av, bk, bv)
            mink = jnp.where(sel, ak, bk)
            minv = jnp.where(sel, av, bv)
            maxk = jnp.where(sel, bk, ak)
            maxv = jnp.where(sel, bv, av)
            up = jnp.full((STILE, 128), False) | ((lo & (k // 128)) == 0)
            kref[lo_s, :] = jnp.where(up, mink, maxk)
            vref[lo_s, :] = jnp.where(up, minv, maxv)
            kref[hi_s, :] = jnp.where(up, maxk, mink)
            vref[hi_s, :] = jnp.where(up, maxv, minv)
            return c

        lax.fori_loop(0, SNT // 2, body, 0)

    k = 2
    while k <= NSORT:
        d = k // 2
        local_ds = []
        while d >= 1:
            if d < 128 * STILE:
                local_ds.append(d)
            else:
                stage_pair(k, d)
            d //= 2
        stage_local(k, local_ds)
        k *= 2

    def wout(t, c):
        out_ref[pl.ds(t * STILE, STILE), :] = vref[pl.ds(t * STILE, STILE), :]
        return c

    lax.fori_loop(0, SNT, wout, 0)


def _bitonic_argsort(keys_padded_2d):
    return pl.pallas_call(
        _bitonic_body,
        out_shape=jax.ShapeDtypeStruct((SROWS, 128), jnp.int32),
        scratch_shapes=[
            pltpu.VMEM((SROWS, 128), jnp.float32),
            pltpu.VMEM((SROWS, 128), jnp.int32),
        ],
    )(keys_padded_2d)


NPOS = 2048


def _cand_body(sorted_hbm, pos_hbm, out_hbm, pos_v, val_v, sem):
    wid = lax.axis_index("s") * 2 + lax.axis_index("c")

    @pl.when(wid < NPOS // CH)
    def _():
        base = wid * CH
        pltpu.sync_copy(pos_hbm.at[pl.ds(base, CH)], pos_v)
        pltpu.async_copy(sorted_hbm.at[pos_v], val_v, sem).wait()
        pltpu.sync_copy(val_v, out_hbm.at[pl.ds(base, CH)])


def _gather_candidates(sorted_flat, pos):
    mesh = plsc.VectorSubcoreMesh(core_axis_name="c", subcore_axis_name="s")
    fn = functools.partial(
        pl.kernel,
        out_type=jax.ShapeDtypeStruct((NPOS,), jnp.int32),
        mesh=mesh,
        compiler_params=pltpu.CompilerParams(use_tc_tiling_on_sc=False),
        scratch_types=[
            pltpu.VMEM((CH,), jnp.int32),
            pltpu.VMEM((CH,), jnp.int32),
            pltpu.SemaphoreType.DMA,
        ],
    )(_cand_body)
    return fn(sorted_flat, pos)


def _candidate_positions():
    key = jax.random.key(42)
    k1, k2 = jax.random.split(key)
    idx1 = jax.random.randint(k1, (1000,), 0, N // 2, dtype=jnp.int32)
    idx2 = jax.random.randint(k2, (1000,), N // 2, N, dtype=jnp.int32)
    pad = jnp.zeros((NPOS - 2000,), jnp.int32)
    return jnp.concatenate([idx1, idx2, pad])


def kernel(node_embeddings, edge_index, num_nodes, fiedler_vector,
           W1, b1, W2, b2, W3, b3):
    f2d = fiedler_vector[:, None]
    p_tab, q_tab = _node_tables(node_embeddings, f2d, W1, b1[None, :])

    h1a, h1b = _edge_gather(p_tab, q_tab, edge_index)
    w2d = jnp.kron(jnp.eye(4, dtype=jnp.float32), W2).astype(jnp.bfloat16)
    b2c = jnp.tile(b2, 4)[:, None]
    w3c = jnp.tile(W3[:, 0], 4)[:, None]
    scores = _mlp_tail(h1a.reshape(E // 4, 4 * H), h1b.reshape(E // 4, 4 * H),
                       w2d, b2c, w3c, b3[None, :])

    keys_padded = jnp.concatenate(
        [fiedler_vector,
         jnp.full((NSORT - N,), jnp.inf, jnp.float32)]).reshape(SROWS, 128)
    sorted_idx = _bitonic_argsort(keys_padded).reshape(NSORT)

    pos = _candidate_positions()
    cand_flat = _gather_candidates(sorted_idx, pos)
    candidate_edges = jnp.stack([cand_flat[:1000], cand_flat[1000:2000]])
    return scores, candidate_edges
